# baseline (speedup 1.0000x reference)
import jax
import jax.numpy as jnp
from jax import lax
from jax.experimental import pallas as pl
from jax.experimental.pallas import tpu as pltpu

V = 100000
D = 10
H = 128
L = 200

NB = 20
RB = V // NB


def _tc_main_body(idx_ref, tbl_ref, w1t_ref, b1_ref, w2_ref, b2_ref,
                  logits_ref, lse_ref, h_ref, m_ref, s_ref):
    j = pl.program_id(0)

    @pl.when(j == 0)
    def _():
        col_iota = lax.broadcasted_iota(jnp.int32, (D, 128), 1)

        def body(i, acc):
            v = idx_ref[i]
            base = pl.multiple_of((v >> 7) * 128, 128)
            c = v & 127
            tile = tbl_ref[:, pl.ds(base, 128)]
            ssv = jnp.sum(tile * tile, axis=0, keepdims=True)
            scale = jnp.where(ssv > 1.0, lax.rsqrt(ssv), 1.0)
            return acc + jnp.where(col_iota == c, tile * scale, 0.0)

        acc = lax.fori_loop(0, L, body, jnp.zeros((D, 128), jnp.float32))
        x = jnp.sum(acc, axis=1, keepdims=True)
        h = lax.dot_general(x, w1t_ref[...], (((0,), (0,)), ((), ())),
                            preferred_element_type=jnp.float32)
        h_ref[...] = jnp.maximum(h + b1_ref[...], 0.0)

    h = h_ref[...]
    logits = lax.dot_general(h, w2_ref[...], (((1,), (1,)), ((), ())),
                             preferred_element_type=jnp.float32)
    logits = logits + b2_ref[0]
    logits_ref[0] = logits

    bm = jnp.max(logits, axis=(0, 1), keepdims=True)

    @pl.when(j == 0)
    def _():
        m_ref[...] = bm
        s_ref[...] = jnp.sum(jnp.exp(logits - bm), axis=(0, 1), keepdims=True)

    @pl.when(j > 0)
    def _():
        m_old = m_ref[...]
        nm = jnp.maximum(m_old, bm)
        s_ref[...] = (s_ref[...] * jnp.exp(m_old - nm)
                      + jnp.sum(jnp.exp(logits - nm), axis=(0, 1), keepdims=True))
        m_ref[...] = nm

    @pl.when(j == NB - 1)
    def _():
        lse_ref[...] = m_ref[...] + jnp.log(s_ref[...])


_tc_main = pl.pallas_call(
    _tc_main_body,
    grid=(NB,),
    in_specs=[
        pl.BlockSpec(memory_space=pltpu.SMEM),
        pl.BlockSpec((D, V), lambda j: (0, 0)),
        pl.BlockSpec((D, H), lambda j: (0, 0)),
        pl.BlockSpec((1, H), lambda j: (0, 0)),
        pl.BlockSpec((RB, H), lambda j: (j, 0)),
        pl.BlockSpec((1, 1, RB), lambda j: (j, 0, 0)),
    ],
    out_specs=[
        pl.BlockSpec((1, 1, RB), lambda j: (j, 0, 0)),
        pl.BlockSpec((1, 1), lambda j: (0, 0)),
    ],
    out_shape=[
        jax.ShapeDtypeStruct((NB, 1, RB), jnp.float32),
        jax.ShapeDtypeStruct((1, 1), jnp.float32),
    ],
    scratch_shapes=[
        pltpu.VMEM((1, H), jnp.float32),
        pltpu.VMEM((1, 1), jnp.float32),
        pltpu.VMEM((1, 1), jnp.float32),
    ],
)


def _tc_sub_body(logits_ref, lse_ref, out_ref):
    out_ref[...] = logits_ref[...] - lse_ref[0, 0]


_tc_sub = pl.pallas_call(
    _tc_sub_body,
    grid=(NB,),
    in_specs=[
        pl.BlockSpec((1, 1, RB), lambda j: (j, 0, 0)),
        pl.BlockSpec(memory_space=pltpu.SMEM),
    ],
    out_specs=pl.BlockSpec((1, 1, RB), lambda j: (j, 0, 0)),
    out_shape=jax.ShapeDtypeStruct((NB, 1, RB), jnp.float32),
)


def kernel(inputs, table, W1, b1, W2, b2):
    logits, lse = _tc_main(
        inputs,
        table.T,
        W1.T,
        b1.reshape(1, H),
        W2,
        b2.reshape(NB, 1, RB),
    )
    out = _tc_sub(logits, lse)
    return out.reshape(1, V)


# speedup vs baseline: 1.3984x; 1.3984x over previous
import jax
import jax.numpy as jnp
from jax import lax
from jax.experimental import pallas as pl
from jax.experimental.pallas import tpu as pltpu

V = 100000
D = 10
H = 128
L = 200

RBP = 5120
NBP = (V + RBP - 1) // RBP
TAIL = V - (NBP - 1) * RBP


def _fused_body(idx_ref, tbl_ref, w1t_ref, b1_ref, w2_ref, b2_ref,
                out_ref, h_ref, m_ref, s_ref):
    j = pl.program_id(0)

    @pl.when(j == 0)
    def _():
        col_iota = lax.broadcasted_iota(jnp.int32, (D, 128), 1)

        def body(i, acc):
            v = idx_ref[i]
            base = pl.multiple_of((v >> 7) * 128, 128)
            c = v & 127
            tile = tbl_ref[:, pl.ds(base, 128)]
            ssv = jnp.sum(tile * tile, axis=0, keepdims=True)
            scale = jnp.where(ssv > 1.0, lax.rsqrt(ssv), 1.0)
            return acc + jnp.where(col_iota == c, tile * scale, 0.0)

        acc = lax.fori_loop(0, L, body, jnp.zeros((D, 128), jnp.float32))
        x = jnp.sum(acc, axis=1, keepdims=True)
        h = lax.dot_general(x, w1t_ref[...], (((0,), (0,)), ((), ())),
                            preferred_element_type=jnp.float32)
        h_ref[...] = jnp.maximum(h + b1_ref[...], 0.0)

    h = h_ref[...]
    logits = lax.dot_general(h, w2_ref[...], (((1,), (1,)), ((), ())),
                             preferred_element_type=jnp.float32)
    logits = logits + b2_ref[...]

    valid = (lax.broadcasted_iota(jnp.int32, (1, RBP), 1) + j * RBP) < V
    lm = jnp.where(valid, logits, -1e30)

    base = pl.multiple_of(j * RBP, 128)

    @pl.when(j < NBP - 1)
    def _():
        out_ref[0, pl.ds(base, RBP)] = logits[0]

    @pl.when(j == NBP - 1)
    def _():
        out_ref[0, pl.ds(base, TAIL)] = logits[0, :TAIL]

    bm = jnp.max(lm, axis=(0, 1), keepdims=True)

    @pl.when(j == 0)
    def _():
        m_ref[...] = bm
        s_ref[...] = jnp.sum(jnp.exp(lm - bm), axis=(0, 1), keepdims=True)

    @pl.when(j > 0)
    def _():
        m_old = m_ref[...]
        nm = jnp.maximum(m_old, bm)
        s_ref[...] = (s_ref[...] * jnp.exp(m_old - nm)
                      + jnp.sum(jnp.exp(lm - nm), axis=(0, 1), keepdims=True))
        m_ref[...] = nm

    @pl.when(j == NBP - 1)
    def _():
        lse = m_ref[0, 0] + jnp.log(s_ref[0, 0])
        out_ref[...] = out_ref[...] - lse


def _make(interpret=False):
    return pl.pallas_call(
        _fused_body,
        grid=(NBP,),
        in_specs=[
            pl.BlockSpec(memory_space=pltpu.SMEM),
            pl.BlockSpec((D, V), lambda j: (0, 0)),
            pl.BlockSpec((D, H), lambda j: (0, 0)),
            pl.BlockSpec((1, H), lambda j: (0, 0)),
            pl.BlockSpec((RBP, H), lambda j: (j, 0)),
            pl.BlockSpec((1, RBP), lambda j: (0, j)),
        ],
        out_specs=pl.BlockSpec((1, V), lambda j: (0, 0)),
        out_shape=jax.ShapeDtypeStruct((1, V), jnp.float32),
        scratch_shapes=[
            pltpu.VMEM((1, H), jnp.float32),
            pltpu.VMEM((1, 1), jnp.float32),
            pltpu.VMEM((1, 1), jnp.float32),
        ],
        interpret=interpret,
    )


def kernel(inputs, table, W1, b1, W2, b2):
    return _make()(
        inputs,
        table.T,
        W1.T,
        b1.reshape(1, H),
        W2,
        b2.reshape(1, V),
    )
